# baseline (speedup 1.0000x reference)
import functools

import jax
import jax.numpy as jnp
from jax.experimental import pallas as pl
from jax.experimental.pallas import tpu as pltpu

M = 1024
K = 128
N = 100000
BN = 2048
GRID = (N + BN - 1) // BN


def _matmul_body(state_ref, mem_ref, out_ref):
    s = state_ref[...]
    m = mem_ref[...]
    out_ref[...] = jax.lax.dot_general(
        s, m, (((1,), (1,)), ((), ())),
        preferred_element_type=jnp.float32,
        precision=jax.lax.Precision.DEFAULT)


def _weights(state_past, memory_past):
    return pl.pallas_call(
        _matmul_body,
        grid=(GRID,),
        in_specs=[
            pl.BlockSpec((M, K), lambda j: (0, 0)),
            pl.BlockSpec((BN, K), lambda j: (j, 0)),
        ],
        out_specs=pl.BlockSpec((M, BN), lambda j: (0, j)),
        out_shape=jax.ShapeDtypeStruct((M, N), jnp.float32),
    )(state_past, memory_past)


def _l2n(x, axis):
    n = jnp.sqrt(jnp.sum(x * x, axis=axis, keepdims=True))
    return x / jnp.maximum(n, 1e-12)


@jax.jit
def kernel(state_past, memory_past):
    sn = _l2n(state_past, 1)
    mn = _l2n(memory_past, 1)
    w = _weights(sn, mn)
    w_xla = jnp.matmul(sn, mn.T)
    index_max = jnp.argsort(-w_xla, axis=-1)
    return (index_max, w)


# speedup vs baseline: 1.2556x; 1.2556x over previous
import functools

import jax
import jax.numpy as jnp
from jax.experimental import pallas as pl
from jax.experimental.pallas import tpu as pltpu

M = 1024
K = 128
N = 100000
BN = 2048
GRID = (N + BN - 1) // BN


def _matmul_body(state_ref, mem_ref, out_ref, key_ref):
    s = state_ref[...]
    m = mem_ref[...]
    w = jax.lax.dot_general(
        s, m, (((1,), (1,)), ((), ())),
        preferred_element_type=jnp.float32,
        precision=jax.lax.Precision.DEFAULT)
    out_ref[...] = w
    b = jax.lax.bitcast_convert_type(w, jnp.int32)
    key_ref[...] = jnp.bitwise_xor(
        jnp.where(b < 0, b, jnp.int32(0x7FFFFFFF) - b),
        jnp.int32(-2147483648))


def _weights_and_keys(state_n, mem_n):
    return pl.pallas_call(
        _matmul_body,
        grid=(GRID,),
        in_specs=[
            pl.BlockSpec((M, K), lambda j: (0, 0)),
            pl.BlockSpec((BN, K), lambda j: (j, 0)),
        ],
        out_specs=[
            pl.BlockSpec((M, BN), lambda j: (0, j)),
            pl.BlockSpec((M, BN), lambda j: (0, j)),
        ],
        out_shape=[
            jax.ShapeDtypeStruct((M, N), jnp.float32),
            jax.ShapeDtypeStruct((M, N), jnp.int32),
        ],
    )(state_n, mem_n)


def _l2n(x, axis):
    n = jnp.sqrt(jnp.sum(x * x, axis=axis, keepdims=True))
    return x / jnp.maximum(n, 1e-12)


@jax.jit
def kernel(state_past, memory_past):
    sn = _l2n(state_past, 1)
    mn = _l2n(memory_past, 1)
    w, v = _weights_and_keys(sn, mn)
    index_max = jnp.argsort(v, axis=-1, stable=True)
    return (index_max, w)


# speedup vs baseline: 1.4686x; 1.1697x over previous
import functools

import jax
import jax.numpy as jnp
from jax.experimental import pallas as pl
from jax.experimental.pallas import tpu as pltpu

M = 1024
K = 128
N = 100000
BN = 2048
GRID = (N + BN - 1) // BN


def _matmul_body(state_ref, mem_ref, out_ref, key_ref):
    s = state_ref[...]
    m = mem_ref[...]
    w = jax.lax.dot_general(
        s, m, (((1,), (1,)), ((), ())),
        preferred_element_type=jnp.float32,
        precision=jax.lax.Precision.DEFAULT)
    out_ref[...] = w
    b = jax.lax.bitcast_convert_type(w, jnp.int32)
    key_ref[...] = jnp.bitwise_xor(
        jnp.where(b < 0, b, jnp.int32(0x7FFFFFFF) - b),
        jnp.int32(-2147483648))


def _weights_and_keys(state_n, mem_n):
    return pl.pallas_call(
        _matmul_body,
        grid=(GRID,),
        in_specs=[
            pl.BlockSpec((M, K), lambda j: (0, 0)),
            pl.BlockSpec((BN, K), lambda j: (j, 0)),
        ],
        out_specs=[
            pl.BlockSpec((M, BN), lambda j: (0, j)),
            pl.BlockSpec((M, BN), lambda j: (0, j)),
        ],
        out_shape=[
            jax.ShapeDtypeStruct((M, N), jnp.float32),
            jax.ShapeDtypeStruct((M, N), jnp.int32),
        ],
    )(state_n, mem_n)


def _l2n(x, axis):
    n = jnp.sqrt(jnp.sum(x * x, axis=axis, keepdims=True))
    return x / jnp.maximum(n, 1e-12)


@jax.jit
def kernel(state_past, memory_past):
    sn = _l2n(state_past, 1)
    mn = _l2n(memory_past, 1)
    w, v = _weights_and_keys(sn, mn)
    index_max = jnp.argsort(v, axis=-1, stable=False)
    return (index_max, w)
